# Initial kernel scaffold; baseline (speedup 1.0000x reference)
import jax
import jax.numpy as jnp
from jax.experimental import pallas as pl


def kernel(z, emb_weight, proj_w, proj_b, l2_scale):
    raise NotImplementedError("write your pallas kernel here")


# speedup vs baseline: 1.1710x; 1.1710x over previous
import functools

import jax
import jax.numpy as jnp
from jax import lax
from jax.experimental import pallas as pl
from jax.experimental.pallas import tpu as pltpu
from jax.experimental.pallas import tpu_sc as plsc

_NUM_EMB = 8192
_DIM = 64
_PAD = 128
_BETA = 0.25
_TOKENS = 9216
_TB = 256
_NB = _TOKENS // _TB
_NW = 32
_BPW = _TOKENS // _NW
_CHUNK = 96
_NCHUNK = _BPW // _CHUNK


def _tc_body(z_ref, emb_ref, pw_ref, pb_ref, idx_ref, qcb_ref, cbn_scr):
    step = pl.program_id(0)

    @pl.when(step == 0)
    def _project():
        qcb = lax.dot_general(
            emb_ref[...], pw_ref[...], (((1,), (1,)), ((), ())),
            preferred_element_type=jnp.float32) + pb_ref[...]
        qcb_ref[...] = jnp.concatenate(
            [qcb, jnp.zeros((_NUM_EMB, _PAD - _DIM), jnp.float32)], axis=1)
        n = jnp.sqrt(jnp.sum(qcb * qcb, axis=1, keepdims=True))
        cbn_scr[...] = qcb / jnp.maximum(n, 1e-12)

    z = z_ref[...]
    zn = z / jnp.maximum(
        jnp.sqrt(jnp.sum(z * z, axis=1, keepdims=True)), 1e-12)
    s = lax.dot_general(
        zn, cbn_scr[...], (((1,), (1,)), ((), ())),
        preferred_element_type=jnp.float32)
    m = jnp.max(s, axis=1, keepdims=True)
    iota = lax.broadcasted_iota(jnp.int32, s.shape, 1)
    masked = jnp.where(s == m, iota, jnp.int32(_NUM_EMB))
    idx_ref[...] = jnp.min(masked, axis=1, keepdims=True)


def _tc_call(zf, emb_weight, proj_w, proj_b2d):
    return pl.pallas_call(
        _tc_body,
        grid=(_NB,),
        in_specs=[
            pl.BlockSpec((_TB, _DIM), lambda i: (i, 0)),
            pl.BlockSpec((_NUM_EMB, _DIM), lambda i: (0, 0)),
            pl.BlockSpec((_DIM, _DIM), lambda i: (0, 0)),
            pl.BlockSpec((1, _DIM), lambda i: (0, 0)),
        ],
        out_specs=[
            pl.BlockSpec((_TB, 1), lambda i: (i, 0)),
            pl.BlockSpec((_NUM_EMB, _PAD), lambda i: (0, 0)),
        ],
        out_shape=[
            jax.ShapeDtypeStruct((_TOKENS, 1), jnp.int32),
            jax.ShapeDtypeStruct((_NUM_EMB, _PAD), jnp.float32),
        ],
        scratch_shapes=[pltpu.VMEM((_NUM_EMB, _DIM), jnp.float32)],
        compiler_params=pltpu.CompilerParams(
            dimension_semantics=("arbitrary",)),
    )(zf, emb_weight, proj_w, proj_b2d)


@functools.lru_cache(maxsize=1)
def _sc_gather_fn():
    mesh = plsc.VectorSubcoreMesh(core_axis_name="c", subcore_axis_name="s")

    @functools.partial(
        pl.kernel,
        mesh=mesh,
        out_type=[
            jax.ShapeDtypeStruct((_TOKENS, _PAD), jnp.float32),
            jax.ShapeDtypeStruct((_NW, 16), jnp.float32),
        ],
        scratch_types=[
            pltpu.VMEM((_BPW,), jnp.int32),
            pltpu.VMEM((_BPW, _PAD), jnp.float32),
            pltpu.VMEM((_BPW * _DIM,), jnp.float32),
            pltpu.VMEM((16,), jnp.float32),
            pltpu.SemaphoreType.DMA,
        ],
    )
    def _sc_gather(qcb_hbm, idx_hbm, zflat_hbm, out_hbm, loss_hbm,
                   idx_v, rows_v, z_v, acc_v, sem):
        c = lax.axis_index("c")
        s = lax.axis_index("s")
        wid = s * 2 + c
        base = wid * _BPW
        pltpu.sync_copy(idx_hbm.at[pl.ds(base, _BPW)], idx_v)
        for j in range(_NCHUNK):
            pltpu.async_copy(
                qcb_hbm.at[idx_v.at[pl.ds(j * _CHUNK, _CHUNK)]],
                rows_v.at[pl.ds(j * _CHUNK, _CHUNK)], sem)
        pltpu.sync_copy(zflat_hbm.at[pl.ds(base * _DIM, _BPW * _DIM)], z_v)
        for j in range(_NCHUNK):
            pltpu.make_async_copy(
                qcb_hbm.at[idx_v.at[pl.ds(j * _CHUNK, _CHUNK)]],
                rows_v.at[pl.ds(j * _CHUNK, _CHUNK)], sem).wait()

        def body(i, acc):
            for k in range(_DIM // 16):
                q = rows_v[i, pl.ds(k * 16, 16)]
                zz = z_v[pl.ds(i * _DIM + k * 16, 16)]
                d = q - zz
                acc = acc + d * d
            return acc

        acc = lax.fori_loop(0, _BPW, body, jnp.zeros((16,), jnp.float32))
        acc_v[...] = acc
        pltpu.sync_copy(rows_v, out_hbm.at[pl.ds(base, _BPW)])
        pltpu.sync_copy(acc_v, loss_hbm.at[wid])

    return _sc_gather


def kernel(z, emb_weight, proj_w, proj_b, l2_scale):
    del l2_scale
    B, T, D = z.shape
    zf = z.reshape(-1, D)
    idx2d, qcb_pad = _tc_call(zf, emb_weight, proj_w, proj_b.reshape(1, D))
    idx = idx2d.reshape(-1)
    quant_pad, loss_rows = _sc_gather_fn()(qcb_pad, idx, zf.reshape(-1))
    quant = quant_pad[:, :_DIM]
    vq_loss = (1.0 + _BETA) * jnp.sum(loss_rows) / zf.size
    return quant.reshape(z.shape), vq_loss, idx.reshape(B, T)


# speedup vs baseline: 1.3469x; 1.1502x over previous
import functools

import jax
import jax.numpy as jnp
from jax import lax
from jax.experimental import pallas as pl
from jax.experimental.pallas import tpu as pltpu
from jax.experimental.pallas import tpu_sc as plsc

_NUM_EMB = 8192
_DIM = 64
_PAD = 128
_BETA = 0.25
_TOKENS = 9216
_TB = 512
_NB = _TOKENS // _TB
_NW = 32
_BPW = _TOKENS // _NW
_CHUNK = 96
_NCHUNK = _BPW // _CHUNK


def _tc_body(z_ref, emb_ref, pw_ref, pb_ref, idx_ref, qcb_ref, cbn_scr,
             iota_scr):
    step = pl.program_id(0)

    @pl.when(step == 0)
    def _project():
        iota_scr[...] = lax.broadcasted_iota(
            jnp.int32, (1, _NUM_EMB), 1).astype(jnp.float32)
        qcb = lax.dot_general(
            emb_ref[...], pw_ref[...], (((1,), (1,)), ((), ())),
            preferred_element_type=jnp.float32) + pb_ref[...]
        qcb_ref[...] = jnp.concatenate(
            [qcb, jnp.zeros((_NUM_EMB, _PAD - _DIM), jnp.float32)], axis=1)
        n = jnp.sqrt(jnp.sum(qcb * qcb, axis=1, keepdims=True))
        cbn_scr[...] = qcb / jnp.maximum(n, 1e-12)

    z = z_ref[...]
    zn = z / jnp.maximum(
        jnp.sqrt(jnp.sum(z * z, axis=1, keepdims=True)), 1e-12)
    s = lax.dot_general(
        zn, cbn_scr[...], (((1,), (1,)), ((), ())),
        preferred_element_type=jnp.float32)
    m = jnp.max(s, axis=1, keepdims=True)
    masked = jnp.where(s == m, iota_scr[...], jnp.float32(_NUM_EMB))
    idx_ref[...] = jnp.min(masked, axis=1, keepdims=True).astype(jnp.int32)


def _tc_call(zf, emb_weight, proj_w, proj_b2d):
    return pl.pallas_call(
        _tc_body,
        grid=(_NB,),
        in_specs=[
            pl.BlockSpec((_TB, _DIM), lambda i: (i, 0)),
            pl.BlockSpec((_NUM_EMB, _DIM), lambda i: (0, 0)),
            pl.BlockSpec((_DIM, _DIM), lambda i: (0, 0)),
            pl.BlockSpec((1, _DIM), lambda i: (0, 0)),
        ],
        out_specs=[
            pl.BlockSpec((_TB, 1), lambda i: (i, 0)),
            pl.BlockSpec((_NUM_EMB, _PAD), lambda i: (0, 0)),
        ],
        out_shape=[
            jax.ShapeDtypeStruct((_TOKENS, 1), jnp.int32),
            jax.ShapeDtypeStruct((_NUM_EMB, _PAD), jnp.float32),
        ],
        scratch_shapes=[pltpu.VMEM((_NUM_EMB, _DIM), jnp.float32),
                        pltpu.VMEM((1, _NUM_EMB), jnp.float32)],
        compiler_params=pltpu.CompilerParams(
            dimension_semantics=("arbitrary",)),
    )(zf, emb_weight, proj_w, proj_b2d)


@functools.lru_cache(maxsize=1)
def _sc_gather_fn():
    mesh = plsc.VectorSubcoreMesh(core_axis_name="c", subcore_axis_name="s")

    @functools.partial(
        pl.kernel,
        mesh=mesh,
        out_type=[
            jax.ShapeDtypeStruct((_TOKENS, _PAD), jnp.float32),
            jax.ShapeDtypeStruct((_NW, 16), jnp.float32),
        ],
        scratch_types=[
            pltpu.VMEM((_BPW,), jnp.int32),
            pltpu.VMEM((_BPW, _PAD), jnp.float32),
            pltpu.VMEM((_BPW * _DIM,), jnp.float32),
            pltpu.VMEM((16,), jnp.float32),
            pltpu.SemaphoreType.DMA,
        ],
    )
    def _sc_gather(qcb_hbm, idx_hbm, zflat_hbm, out_hbm, loss_hbm,
                   idx_v, rows_v, z_v, acc_v, sem):
        c = lax.axis_index("c")
        s = lax.axis_index("s")
        wid = s * 2 + c
        base = wid * _BPW
        pltpu.sync_copy(idx_hbm.at[pl.ds(base, _BPW)], idx_v)
        for j in range(_NCHUNK):
            pltpu.async_copy(
                qcb_hbm.at[idx_v.at[pl.ds(j * _CHUNK, _CHUNK)]],
                rows_v.at[pl.ds(j * _CHUNK, _CHUNK)], sem)
        pltpu.sync_copy(zflat_hbm.at[pl.ds(base * _DIM, _BPW * _DIM)], z_v)
        for j in range(_NCHUNK):
            pltpu.make_async_copy(
                qcb_hbm.at[idx_v.at[pl.ds(j * _CHUNK, _CHUNK)]],
                rows_v.at[pl.ds(j * _CHUNK, _CHUNK)], sem).wait()

        def body(i, acc):
            for k in range(_DIM // 16):
                q = rows_v[i, pl.ds(k * 16, 16)]
                zz = z_v[pl.ds(i * _DIM + k * 16, 16)]
                d = q - zz
                acc = acc + d * d
            return acc

        acc = lax.fori_loop(0, _BPW, body, jnp.zeros((16,), jnp.float32))
        acc_v[...] = acc
        pltpu.sync_copy(rows_v, out_hbm.at[pl.ds(base, _BPW)])
        pltpu.sync_copy(acc_v, loss_hbm.at[wid])

    return _sc_gather


def kernel(z, emb_weight, proj_w, proj_b, l2_scale):
    del l2_scale
    B, T, D = z.shape
    zf = z.reshape(-1, D)
    idx2d, qcb_pad = _tc_call(zf, emb_weight, proj_w, proj_b.reshape(1, D))
    idx = idx2d.reshape(-1)
    quant_pad, loss_rows = _sc_gather_fn()(qcb_pad, idx, zf.reshape(-1))
    quant = quant_pad[:, :_DIM]
    vq_loss = (1.0 + _BETA) * jnp.sum(loss_rows) / zf.size
    return quant.reshape(z.shape), vq_loss, idx.reshape(B, T)


# speedup vs baseline: 1.4359x; 1.0661x over previous
import functools

import jax
import jax.numpy as jnp
from jax import lax
from jax.experimental import pallas as pl
from jax.experimental.pallas import tpu as pltpu
from jax.experimental.pallas import tpu_sc as plsc

_NUM_EMB = 8192
_DIM = 64
_PAD = 128
_BETA = 0.25
_TOKENS = 9216
_TB = 128
_NB = _TOKENS // _TB
_NW = 32
_BPW = _TOKENS // _NW
_CHUNK = 96
_NCHUNK = _BPW // _CHUNK


def _tc_body(z_ref, emb_ref, pw_ref, pbr_ref, pbc_ref, idx_ref, qcb_ref,
             cbnT_scr, iota_scr):
    step = pl.program_id(0)

    @pl.when(step == 0)
    def _project():
        iota_scr[...] = lax.broadcasted_iota(
            jnp.int32, (1, _NUM_EMB), 1).astype(jnp.float32)
        emb = emb_ref[...]
        qcb = lax.dot_general(
            emb, pw_ref[...], (((1,), (1,)), ((), ())),
            preferred_element_type=jnp.float32) + pbr_ref[...]
        qcb_ref[...] = jnp.concatenate(
            [qcb, jnp.zeros((_NUM_EMB, _PAD - _DIM), jnp.float32)], axis=1)
        qcbT = lax.dot_general(
            pw_ref[...], emb, (((1,), (1,)), ((), ())),
            preferred_element_type=jnp.float32) + pbc_ref[...]
        n2 = jnp.sum(qcbT * qcbT, axis=0, keepdims=True)
        r = 1.0 / jnp.maximum(jnp.sqrt(n2), 1e-12)
        cbnT_scr[...] = qcbT * r

    z = z_ref[...]
    s = lax.dot_general(
        z, cbnT_scr[...], (((1,), (0,)), ((), ())),
        preferred_element_type=jnp.float32)
    bv = lax.slice(s, (0, 0), (_TB, 128))
    bi = jnp.broadcast_to(iota_scr[:, 0:128], (_TB, 128))
    for j in range(1, _NUM_EMB // 128):
        sj = lax.slice(s, (0, j * 128), (_TB, (j + 1) * 128))
        ij = jnp.broadcast_to(iota_scr[:, j * 128:(j + 1) * 128], (_TB, 128))
        gt = sj > bv
        bv = jnp.maximum(bv, sj)
        bi = jnp.where(gt, ij, bi)
    m = jnp.max(bv, axis=1, keepdims=True)
    masked = jnp.where(bv == m, bi, jnp.float32(_NUM_EMB))
    idx_ref[...] = jnp.min(masked, axis=1, keepdims=True).astype(jnp.int32)


def _tc_call(zf, emb_weight, proj_w, proj_b2d):
    return pl.pallas_call(
        _tc_body,
        grid=(_NB,),
        in_specs=[
            pl.BlockSpec((_TB, _DIM), lambda i: (i, 0)),
            pl.BlockSpec((_NUM_EMB, _DIM), lambda i: (0, 0)),
            pl.BlockSpec((_DIM, _DIM), lambda i: (0, 0)),
            pl.BlockSpec((1, _DIM), lambda i: (0, 0)),
            pl.BlockSpec((_DIM, 1), lambda i: (0, 0)),
        ],
        out_specs=[
            pl.BlockSpec((_TB, 1), lambda i: (i, 0)),
            pl.BlockSpec((_NUM_EMB, _PAD), lambda i: (0, 0)),
        ],
        out_shape=[
            jax.ShapeDtypeStruct((_TOKENS, 1), jnp.int32),
            jax.ShapeDtypeStruct((_NUM_EMB, _PAD), jnp.float32),
        ],
        scratch_shapes=[pltpu.VMEM((_DIM, _NUM_EMB), jnp.float32),
                        pltpu.VMEM((1, _NUM_EMB), jnp.float32)],
        compiler_params=pltpu.CompilerParams(
            dimension_semantics=("arbitrary",)),
    )(zf, emb_weight, proj_w, proj_b2d, proj_b2d.reshape(_DIM, 1))


@functools.lru_cache(maxsize=1)
def _sc_gather_fn():
    mesh = plsc.VectorSubcoreMesh(core_axis_name="c", subcore_axis_name="s")

    @functools.partial(
        pl.kernel,
        mesh=mesh,
        out_type=[
            jax.ShapeDtypeStruct((_TOKENS, _PAD), jnp.float32),
            jax.ShapeDtypeStruct((_NW, 16), jnp.float32),
        ],
        scratch_types=[
            pltpu.VMEM((_BPW,), jnp.int32),
            pltpu.VMEM((_BPW, _PAD), jnp.float32),
            pltpu.VMEM((_BPW * _DIM,), jnp.float32),
            pltpu.VMEM((16,), jnp.float32),
            pltpu.SemaphoreType.DMA,
        ],
    )
    def _sc_gather(qcb_hbm, idx_hbm, zflat_hbm, out_hbm, loss_hbm,
                   idx_v, rows_v, z_v, acc_v, sem):
        c = lax.axis_index("c")
        s = lax.axis_index("s")
        wid = s * 2 + c
        base = wid * _BPW
        pltpu.sync_copy(idx_hbm.at[pl.ds(base, _BPW)], idx_v)
        for j in range(_NCHUNK):
            pltpu.async_copy(
                qcb_hbm.at[idx_v.at[pl.ds(j * _CHUNK, _CHUNK)]],
                rows_v.at[pl.ds(j * _CHUNK, _CHUNK)], sem)
        pltpu.sync_copy(zflat_hbm.at[pl.ds(base * _DIM, _BPW * _DIM)], z_v)
        for j in range(_NCHUNK):
            pltpu.make_async_copy(
                qcb_hbm.at[idx_v.at[pl.ds(j * _CHUNK, _CHUNK)]],
                rows_v.at[pl.ds(j * _CHUNK, _CHUNK)], sem).wait()

        def body(i, acc):
            for k in range(_DIM // 16):
                q = rows_v[i, pl.ds(k * 16, 16)]
                zz = z_v[pl.ds(i * _DIM + k * 16, 16)]
                d = q - zz
                acc = acc + d * d
            return acc

        acc = lax.fori_loop(0, _BPW, body, jnp.zeros((16,), jnp.float32))
        acc_v[...] = acc
        pltpu.sync_copy(rows_v, out_hbm.at[pl.ds(base, _BPW)])
        pltpu.sync_copy(acc_v, loss_hbm.at[wid])

    return _sc_gather


def kernel(z, emb_weight, proj_w, proj_b, l2_scale):
    del l2_scale
    B, T, D = z.shape
    zf = z.reshape(-1, D)
    idx2d, qcb_pad = _tc_call(zf, emb_weight, proj_w, proj_b.reshape(1, D))
    idx = idx2d.reshape(-1)
    quant_pad, loss_rows = _sc_gather_fn()(qcb_pad, idx, zf.reshape(-1))
    quant = quant_pad[:, :_DIM]
    vq_loss = (1.0 + _BETA) * jnp.sum(loss_rows) / zf.size
    return quant.reshape(z.shape), vq_loss, idx.reshape(B, T)
